# baseline (speedup 1.0000x reference)
import functools

import jax
import jax.numpy as jnp
from jax import lax
from jax.experimental import pallas as pl
from jax.experimental.pallas import tpu as pltpu
from jax.experimental.pallas import tpu_sc as plsc

MAXR = 64
CTR = 67
NC, NS = 2, 16
NW = NC * NS
ZP, YP, XP = 144, 136, 72
SLAB = YP * XP
G3 = ZP * SLAB
K = 2448
CH = G3 // (NW * K)
BW4 = 256


def _table_body(w4_ref, b4_ref, a2_ref, ab2_ref, out_ref):
    out_ref[...] = (
        jnp.dot(w4_ref[...], a2_ref[...], preferred_element_type=jnp.float32)
        + jnp.dot(b4_ref[...], ab2_ref[...], preferred_element_type=jnp.float32))


def _eidx_body(sent, idx_ref, out_ref):
    z = 8 * pl.program_id(0) + lax.broadcasted_iota(jnp.int32, (8, SLAB), 0)
    p = lax.broadcasted_iota(jnp.int32, (8, SLAB), 1)
    y = p // XP
    x = p % XP
    r2 = (z - CTR) ** 2 + (y - CTR) ** 2 + x * x
    idx = idx_ref[...]
    valid = (idx >= 0) & (r2 < MAXR * MAXR)
    out_ref[...] = jnp.where(valid, idx, sent)


def _gather_body(table_hbm, eidx_hbm, out_hbm, i0, i1, r0, r1,
                 si0, si1, sg, so0, so1):
    wid = lax.axis_index("s") * NC + lax.axis_index("c")
    w0 = wid * CH
    idxb, rowb = [i0, i1], [r0, r1]
    sib, sob = [si0, si1], [so0, so1]
    hidx = [None, None]
    hout = [None, None]
    hidx[0] = pltpu.async_copy(eidx_hbm.at[pl.ds(w0 * K, K)], idxb[0], sib[0])
    for i in range(CH):
        p = i % 2
        hidx[p].wait()
        pltpu.async_copy(table_hbm.at[idxb[p]], rowb[p], sg).wait()
        if i + 1 < CH:
            hidx[1 - p] = pltpu.async_copy(
                eidx_hbm.at[pl.ds((w0 + i + 1) * K, K)], idxb[1 - p], sib[1 - p])
        if hout[p] is not None:
            hout[p].wait()
        hout[p] = pltpu.async_copy(
            rowb[p], out_hbm.at[pl.ds((w0 + i) * K, K)], sob[p])
    hout[(CH - 1) % 2].wait()
    if hout[CH % 2] is not None:
        hout[CH % 2].wait()


def _extract_body(v_ref, out_ref):
    b = pl.program_id(1)
    l_io = lax.broadcasted_iota(jnp.int32, (1152, 136), 0)
    m_io = lax.broadcasted_iota(jnp.int32, (1152, 136), 1)
    tgt = 16 * (m_io // 2) + 2 * b + (m_io % 2)
    sel = (l_io == tgt).astype(jnp.float32)
    res = jnp.dot(v_ref[...], sel, preferred_element_type=jnp.float32)
    out_ref[0, 0] = res[:135, :]


def kernel(input, weight, bias, grid3d_index):
    wc = weight.shape[0]
    r_pad = 2048 * ((wc + 1 + 2047) // 2048)
    nrow4 = r_pad // 8
    sent = wc
    f32, i32 = jnp.float32, jnp.int32

    inp_t = input.astype(f32).T
    eye2 = jnp.eye(2, dtype=f32)
    a_mat = (inp_t[:, None, :, None] * eye2[None, :, None, :]).reshape(32, 16)
    ab_mat = jnp.tile(eye2, (1, 8))
    a2 = jnp.kron(jnp.eye(8, dtype=f32), a_mat)
    ab2 = jnp.kron(jnp.eye(8, dtype=f32), ab_mat)

    wflat = weight.astype(f32).reshape(-1)
    w4 = jnp.concatenate(
        [wflat, jnp.zeros(((r_pad - wc) * 32,), f32)]).reshape(nrow4, 256)
    bflat = bias.astype(f32).reshape(-1)
    b4 = jnp.concatenate(
        [bflat, jnp.zeros(((r_pad - wc) * 2,), f32)]).reshape(nrow4, 16)

    table2 = pl.pallas_call(
        _table_body,
        grid=(nrow4 // BW4,),
        in_specs=[
            pl.BlockSpec((BW4, 256), lambda r: (r, 0)),
            pl.BlockSpec((BW4, 16), lambda r: (r, 0)),
            pl.BlockSpec((256, 128), lambda r: (0, 0)),
            pl.BlockSpec((16, 128), lambda r: (0, 0)),
        ],
        out_specs=pl.BlockSpec((BW4, 128), lambda r: (r, 0)),
        out_shape=jax.ShapeDtypeStruct((nrow4, 128), f32),
    )(w4, b4, a2, ab2)
    t16 = table2.reshape(r_pad, 16)

    gpad = jnp.pad(grid3d_index.astype(i32),
                   ((0, ZP - 135), (0, YP - 135), (0, XP - 68)),
                   constant_values=-1).reshape(ZP, SLAB)
    eidx2 = pl.pallas_call(
        functools.partial(_eidx_body, sent),
        grid=(ZP // 8,),
        in_specs=[pl.BlockSpec((8, SLAB), lambda r: (r, 0))],
        out_specs=pl.BlockSpec((8, SLAB), lambda r: (r, 0)),
        out_shape=jax.ShapeDtypeStruct((ZP, SLAB), i32),
    )(gpad)
    eflat = eidx2.reshape(G3)

    mesh = plsc.VectorSubcoreMesh(
        core_axis_name="c", subcore_axis_name="s",
        num_cores=NC, num_subcores=NS)
    out_t = pl.kernel(
        _gather_body,
        out_type=jax.ShapeDtypeStruct((G3, 16), f32),
        mesh=mesh,
        compiler_params=pltpu.CompilerParams(use_tc_tiling_on_sc=False),
        scratch_types=[
            pltpu.VMEM((K,), i32), pltpu.VMEM((K,), i32),
            pltpu.VMEM((K, 16), f32), pltpu.VMEM((K, 16), f32),
            pltpu.SemaphoreType.DMA, pltpu.SemaphoreType.DMA,
            pltpu.SemaphoreType.DMA, pltpu.SemaphoreType.DMA,
            pltpu.SemaphoreType.DMA,
        ],
    )(t16, eflat)

    slabs = out_t.reshape(ZP * YP, 1152)
    out136 = pl.pallas_call(
        _extract_body,
        grid=(135, 8),
        in_specs=[pl.BlockSpec((YP, 1152), lambda z, b: (z, 0))],
        out_specs=pl.BlockSpec((1, 1, 135, 136), lambda z, b: (b, z, 0, 0)),
        out_shape=jax.ShapeDtypeStruct((8, 135, 135, 136), f32),
    )(slabs)

    return out136.reshape(8, 135, 135, 68, 2)


# speedup vs baseline: 1.0252x; 1.0252x over previous
import functools

import jax
import jax.numpy as jnp
from jax import lax
from jax.experimental import pallas as pl
from jax.experimental.pallas import tpu as pltpu
from jax.experimental.pallas import tpu_sc as plsc

MAXR = 64
CTR = 67
NC, NS = 2, 16
NW = NC * NS
ZP, YP, XP = 144, 136, 72
SLAB = YP * XP
G3 = ZP * SLAB
K = 1224
KQ = K // 8
CH = G3 // (NW * K)
NSUB = 3
KS = K // NSUB
BW4 = 256
RT = G3 * 16 // 128


def _table_body(w4_ref, b4_ref, a2_ref, ab2_ref, out_ref):
    out_ref[...] = (
        jnp.dot(w4_ref[...], a2_ref[...], preferred_element_type=jnp.float32)
        + jnp.dot(b4_ref[...], ab2_ref[...], preferred_element_type=jnp.float32))


def _eidx_body(sent, idx_ref, out_ref):
    z = 8 * pl.program_id(0) + lax.broadcasted_iota(jnp.int32, (8, SLAB), 0)
    p = lax.broadcasted_iota(jnp.int32, (8, SLAB), 1)
    y = p // XP
    x = p % XP
    r2 = (z - CTR) ** 2 + (y - CTR) ** 2 + x * x
    idx = idx_ref[...]
    valid = (idx >= 0) & (r2 < MAXR * MAXR)
    out_ref[...] = jnp.where(valid, idx, sent)


def _gather_body(table_hbm, eidx_hbm, out_hbm, i0, i1, r0, r1, q0, q1,
                 si0, si1, sg, so0, so1):
    wid = lax.axis_index("s") * NC + lax.axis_index("c")
    w0 = wid * CH
    idxb, rowb, pkb = [i0, i1], [r0, r1], [q0, q1]
    sib, sob = [si0, si1], [so0, so1]
    hidx = [None, None]
    hout = [None, None]
    hidx[0] = pltpu.async_copy(eidx_hbm.at[pl.ds(w0 * K, K)], idxb[0], sib[0])
    for i in range(CH):
        p = i % 2
        hidx[p].wait()
        hg = []
        for s in range(NSUB):
            hg.append(pltpu.async_copy(
                table_hbm.at[idxb[p].at[pl.ds(s * KS, KS)]],
                rowb[p].at[pl.ds(s * KS, KS)], sg))
        if i + 1 < CH:
            hidx[1 - p] = pltpu.async_copy(
                eidx_hbm.at[pl.ds((w0 + i + 1) * K, K)], idxb[1 - p], sib[1 - p])
        for h in hg:
            h.wait()
        if hout[p] is not None:
            hout[p].wait()

        def repack(r, _, rv=rowb[p], qv=pkb[p]):
            for s in range(8):
                qv[r, pl.ds(16 * s, 16)] = rv[8 * r + s, :]
            return 0

        lax.fori_loop(0, KQ, repack, 0, unroll=2)
        hout[p] = pltpu.async_copy(
            pkb[p], out_hbm.at[pl.ds((w0 + i) * KQ, KQ)], sob[p])
    hout[(CH - 1) % 2].wait()
    if hout[CH % 2] is not None:
        hout[CH % 2].wait()


def _extract_body(v_ref, out_ref, pmat, smat):
    z = pl.program_id(0)

    @pl.when(z == 0)
    def _init():
        y_io = lax.broadcasted_iota(jnp.int32, (YP, 9 * YP), 0)
        r_io = lax.broadcasted_iota(jnp.int32, (YP, 9 * YP), 1)
        for t in range(9):
            pmat[t, :, :] = (r_io == 9 * y_io + t).astype(jnp.float32)
        l_io = lax.broadcasted_iota(jnp.int32, (1152, 1088), 0)
        c_io = lax.broadcasted_iota(jnp.int32, (1152, 1088), 1)
        b_c = c_io // 136
        m_c = c_io % 136
        smat[...] = (l_io == 16 * (m_c // 2) + 2 * b_c + (m_c % 2)
                     ).astype(jnp.float32)

    v = v_ref[...]
    merged = []
    for t in range(9):
        merged.append(jnp.dot(pmat[t, :, :], v,
                              preferred_element_type=jnp.float32))
    v9 = jnp.concatenate(merged, axis=1)
    res = jnp.dot(v9, smat[...], preferred_element_type=jnp.float32)
    for b in range(8):
        out_ref[b, 0, :, :] = res[:135, 136 * b:136 * (b + 1)]


def kernel(input, weight, bias, grid3d_index):
    wc = weight.shape[0]
    r_pad = 2048 * ((wc + 1 + 2047) // 2048)
    nrow4 = r_pad // 8
    sent = wc
    f32, i32 = jnp.float32, jnp.int32

    inp_t = input.astype(f32).T
    eye2 = jnp.eye(2, dtype=f32)
    a_mat = (inp_t[:, None, :, None] * eye2[None, :, None, :]).reshape(32, 16)
    ab_mat = jnp.tile(eye2, (1, 8))
    a2 = jnp.kron(jnp.eye(8, dtype=f32), a_mat)
    ab2 = jnp.kron(jnp.eye(8, dtype=f32), ab_mat)

    wflat = weight.astype(f32).reshape(-1)
    w4 = jnp.concatenate(
        [wflat, jnp.zeros(((r_pad - wc) * 32,), f32)]).reshape(nrow4, 256)
    bflat = bias.astype(f32).reshape(-1)
    b4 = jnp.concatenate(
        [bflat, jnp.zeros(((r_pad - wc) * 2,), f32)]).reshape(nrow4, 16)

    table2 = pl.pallas_call(
        _table_body,
        grid=(nrow4 // BW4,),
        in_specs=[
            pl.BlockSpec((BW4, 256), lambda r: (r, 0)),
            pl.BlockSpec((BW4, 16), lambda r: (r, 0)),
            pl.BlockSpec((256, 128), lambda r: (0, 0)),
            pl.BlockSpec((16, 128), lambda r: (0, 0)),
        ],
        out_specs=pl.BlockSpec((BW4, 128), lambda r: (r, 0)),
        out_shape=jax.ShapeDtypeStruct((nrow4, 128), f32),
    )(w4, b4, a2, ab2)
    t16 = table2.reshape(r_pad, 16)

    gpad = jnp.pad(grid3d_index.astype(i32),
                   ((0, ZP - 135), (0, YP - 135), (0, XP - 68)),
                   constant_values=-1).reshape(ZP, SLAB)
    eidx2 = pl.pallas_call(
        functools.partial(_eidx_body, sent),
        grid=(ZP // 8,),
        in_specs=[pl.BlockSpec((8, SLAB), lambda r: (r, 0))],
        out_specs=pl.BlockSpec((8, SLAB), lambda r: (r, 0)),
        out_shape=jax.ShapeDtypeStruct((ZP, SLAB), i32),
    )(gpad)
    eflat = eidx2.reshape(G3)

    mesh = plsc.VectorSubcoreMesh(
        core_axis_name="c", subcore_axis_name="s",
        num_cores=NC, num_subcores=NS)
    out_t = pl.kernel(
        _gather_body,
        out_type=jax.ShapeDtypeStruct((RT, 128), f32),
        mesh=mesh,
        compiler_params=pltpu.CompilerParams(use_tc_tiling_on_sc=False),
        scratch_types=[
            pltpu.VMEM((K,), i32), pltpu.VMEM((K,), i32),
            pltpu.VMEM((K, 16), f32), pltpu.VMEM((K, 16), f32),
            pltpu.VMEM((KQ, 128), f32), pltpu.VMEM((KQ, 128), f32),
            pltpu.SemaphoreType.DMA, pltpu.SemaphoreType.DMA,
            pltpu.SemaphoreType.DMA, pltpu.SemaphoreType.DMA,
            pltpu.SemaphoreType.DMA,
        ],
    )(t16, eflat)

    out136 = pl.pallas_call(
        _extract_body,
        grid=(135,),
        in_specs=[pl.BlockSpec((1224, 128), lambda z: (z, 0))],
        out_specs=pl.BlockSpec((8, 1, 135, 136), lambda z: (0, z, 0, 0)),
        out_shape=jax.ShapeDtypeStruct((8, 135, 135, 136), f32),
        scratch_shapes=[
            pltpu.VMEM((9, YP, 9 * YP), f32),
            pltpu.VMEM((1152, 1088), f32),
        ],
    )(out_t)

    return out136.reshape(8, 135, 135, 68, 2)


# speedup vs baseline: 1.2771x; 1.2457x over previous
import functools

import jax
import jax.numpy as jnp
from jax import lax
from jax.experimental import pallas as pl
from jax.experimental.pallas import tpu as pltpu
from jax.experimental.pallas import tpu_sc as plsc

MAXR = 64
NC, NS = 2, 16
K = 2048
CPW = 19
G_PAD = NC * NS * K * CPW
BW = 512
BG = 4096


def _table_body(wc, w2_ref, b_ref, a_ref, ab_ref, out_ref):
    r = pl.program_id(0)
    val = jnp.dot(w2_ref[...], a_ref[...], preferred_element_type=jnp.float32,
                  precision=lax.Precision.HIGHEST)
    val += jnp.dot(b_ref[...], ab_ref[...], preferred_element_type=jnp.float32,
                   precision=lax.Precision.HIGHEST)
    row = r * BW + lax.broadcasted_iota(jnp.int32, (BW, 16), 0)
    out_ref[...] = jnp.where(row < wc, val, 0.0)


def _eidx_body(sent, idx_ref, out_ref):
    z = pl.program_id(0)
    bzm = idx_ref.shape[1]
    c = bzm // 2
    yy = lax.broadcasted_iota(jnp.int32, idx_ref.shape, 1)
    xx = lax.broadcasted_iota(jnp.int32, idx_ref.shape, 2)
    r2 = (z - c) ** 2 + (yy - c) ** 2 + xx * xx
    idx = idx_ref[...]
    valid = (idx >= 0) & (r2 < MAXR * MAXR)
    out_ref[...] = jnp.where(valid, idx, sent)


def _transpose_body(t_ref, out_ref):
    blk = t_ref[...]
    for b in range(8):
        out_ref[b, :, :] = blk[:, 2 * b:2 * b + 2]


def _gather_body(table_hbm, eidx_hbm, out_hbm, idx_v, rows_v, sem):
    wid = lax.axis_index("s") * NC + lax.axis_index("c")
    for i in range(CPW):
        base = (wid * CPW + i) * K
        pltpu.sync_copy(eidx_hbm.at[pl.ds(base, K)], idx_v)
        pltpu.async_copy(table_hbm.at[idx_v], rows_v, sem).wait()
        pltpu.sync_copy(rows_v, out_hbm.at[pl.ds(base, K)])


def kernel(input, weight, bias, grid3d_index):
    wc = weight.shape[0]
    r_pad = BW * ((wc + 1 + BW - 1) // BW)
    sent = wc
    bzm, _, bzxm = grid3d_index.shape
    g_n = bzm * bzm * bzxm
    f32 = jnp.float32

    inp_t = input.astype(f32).T
    eye2 = jnp.eye(2, dtype=f32)
    a_mat = (inp_t[:, None, :, None] * eye2[None, :, None, :]).reshape(32, 16)
    ab_mat = jnp.tile(eye2, (1, 8))
    w2 = weight.astype(f32).reshape(wc, 32)

    table = pl.pallas_call(
        functools.partial(_table_body, wc),
        grid=(r_pad // BW,),
        in_specs=[
            pl.BlockSpec((BW, 32), lambda r: (r, 0)),
            pl.BlockSpec((BW, 2), lambda r: (r, 0)),
            pl.BlockSpec((32, 16), lambda r: (0, 0)),
            pl.BlockSpec((2, 16), lambda r: (0, 0)),
        ],
        out_specs=pl.BlockSpec((BW, 16), lambda r: (r, 0)),
        out_shape=jax.ShapeDtypeStruct((r_pad, 16), f32),
    )(w2, bias.astype(f32), a_mat, ab_mat)

    gidx = grid3d_index.astype(jnp.int32)
    eidx3 = pl.pallas_call(
        functools.partial(_eidx_body, sent),
        grid=(bzm,),
        in_specs=[pl.BlockSpec((1, bzm, bzxm), lambda z: (z, 0, 0))],
        out_specs=pl.BlockSpec((1, bzm, bzxm), lambda z: (z, 0, 0)),
        out_shape=jax.ShapeDtypeStruct((bzm, bzm, bzxm), jnp.int32),
    )(gidx)
    eidx = jnp.concatenate(
        [eidx3.reshape(g_n), jnp.full((G_PAD - g_n,), sent, jnp.int32)])

    mesh = plsc.VectorSubcoreMesh(
        core_axis_name="c", subcore_axis_name="s",
        num_cores=NC, num_subcores=NS)
    out_t = pl.kernel(
        _gather_body,
        out_type=jax.ShapeDtypeStruct((G_PAD, 16), f32),
        mesh=mesh,
        compiler_params=pltpu.CompilerParams(use_tc_tiling_on_sc=False),
        scratch_types=[
            pltpu.VMEM((K,), jnp.int32),
            pltpu.VMEM((K, 16), f32),
            pltpu.SemaphoreType.DMA,
        ],
    )(table, eidx)

    out_full = pl.pallas_call(
        _transpose_body,
        grid=(G_PAD // BG,),
        in_specs=[pl.BlockSpec((BG, 16), lambda g: (g, 0))],
        out_specs=pl.BlockSpec((8, BG, 2), lambda g: (0, g, 0)),
        out_shape=jax.ShapeDtypeStruct((8, G_PAD, 2), f32),
    )(out_t)

    return out_full[:, :g_n, :].reshape(8, bzm, bzm, bzxm, 2)
